# baseline (speedup 1.0000x reference)
import functools

import jax
import jax.numpy as jnp
from jax import lax
from jax.experimental import pallas as pl
from jax.experimental.pallas import tpu as pltpu
from jax.experimental.pallas import tpu_sc as plsc

_C = 2
_ROWS = 10

_NC, _NS, _L = 2, 16, 16
_CCH = 8
_RBANDS = 4
_NG = 19
_NPAD = _NG * _L


def _sc_crop_split(data, rois):
    cc, h, w, n = data.shape
    wch = w // _CCH
    rh = h // _RBANDS
    rt = jnp.pad(rois.T, ((0, 0), (0, _NPAD - n)))
    mesh = plsc.VectorSubcoreMesh(
        core_axis_name="c", subcore_axis_name="s",
        num_cores=_NC, num_subcores=_NS)

    @functools.partial(
        pl.kernel,
        out_type=jax.ShapeDtypeStruct((h, w, n), data.dtype),
        mesh=mesh,
        compiler_params=pltpu.CompilerParams(
            use_tc_tiling_on_sc=False, needs_layout_passes=False),
        scratch_types=[
            pltpu.VMEM((_NPAD,), jnp.float32),
            pltpu.VMEM((_NPAD,), jnp.float32),
            pltpu.VMEM((_NPAD,), jnp.float32),
            pltpu.VMEM((_NPAD,), jnp.float32),
            pltpu.VMEM((_NPAD,), jnp.float32),
            pltpu.VMEM((_NPAD,), jnp.float32),
            pltpu.VMEM((wch * _NPAD,), jnp.int32),
            pltpu.VMEM((wch * _NPAD,), jnp.int32),
            pltpu.VMEM((cc, wch, n), jnp.float32),
            pltpu.VMEM((wch, n), jnp.float32),
        ],
    )
    def sck(rt_hbm, data_hbm, out_hbm,
            x1v, y1v, x2v, y2v, xmv, ymv, sxi, ixi, dv, ov):
        cid = lax.axis_index("c")
        sid = lax.axis_index("s")
        wid = sid * _NC + cid
        w0 = (wid % _CCH) * wch
        h0 = (wid // _CCH) * rh

        pltpu.sync_copy(rt_hbm.at[0], x1v)
        pltpu.sync_copy(rt_hbm.at[1], y1v)
        pltpu.sync_copy(rt_hbm.at[2], x2v)
        pltpu.sync_copy(rt_hbm.at[3], y2v)

        def initg(g, carry):
            sl = pl.ds(g * _L, _L)
            x1g = x1v[sl]
            x2g = x2v[sl]
            xmv[sl] = x1g + (x2g - x1g) * 0.5
            y1g = y1v[sl]
            y2g = y2v[sl]
            ymv[sl] = y1g + (y2g - y1g) * 0.5
            return carry

        lax.fori_loop(0, _NG, initg, 0)

        i32_one = jnp.ones((_L,), jnp.int32)
        i32_zero = jnp.zeros((_L,), jnp.int32)

        def initw(t, carry):
            wi = t // _NG
            g = t - wi * _NG
            wfv = jnp.zeros((_L,), jnp.float32) + (w0 + wi).astype(jnp.float32)
            sl = pl.ds(g * _L, _L)
            fl = pl.ds(wi * _NPAD + g * _L, _L)
            sxi[fl] = jnp.where(wfv >= xmv[sl], i32_one, i32_zero)
            ixw = (wfv >= x1v[sl]) & (wfv <= x2v[sl])
            ixi[fl] = jnp.where(ixw, i32_one, i32_zero)
            return carry

        lax.fori_loop(0, wch * _NG, initw, 0)

        def row(r, carry):
            hr = h0 + r
            for p in range(4):
                pltpu.sync_copy(data_hbm.at[p, hr, pl.ds(w0, wch), :],
                                dv.at[p])
            hfv = jnp.zeros((_L,), jnp.float32) + hr.astype(jnp.float32)

            for g in range(_NG - 1):
                sl = pl.ds(g * _L, _L)
                sy = hfv >= ymv[sl]
                iy = (hfv >= y1v[sl]) & (hfv <= y2v[sl])
                for wi in range(wch):
                    fl = pl.ds(wi * _NPAD + g * _L, _L)
                    sx = sxi[fl] != 0
                    d0 = dv[0, wi, sl]
                    d1 = dv[1, wi, sl]
                    d2 = dv[2, wi, sl]
                    d3 = dv[3, wi, sl]
                    low = jnp.where(sx, d1, d0)
                    high = jnp.where(sx, d3, d2)
                    sel = jnp.where(sy, high, low)
                    ins = (ixi[fl] != 0) & iy
                    ov[wi, sl] = jnp.where(ins, sel, jnp.float32(0.0))

            gt = _NG - 1
            slt = pl.ds(gt * _L, _L)
            lane = gt * _L + lax.iota(jnp.int32, _L)
            lmask = lane < n
            syt = hfv >= ymv[slt]
            sy2 = jnp.where(syt, i32_one + i32_one, i32_zero)
            iyt = (hfv >= y1v[slt]) & (hfv <= y2v[slt])
            for wi in range(wch):
                flt = pl.ds(wi * _NPAD + gt * _L, _L)
                plane = sxi[flt] + sy2
                wv = jnp.full((_L,), wi, jnp.int32)
                vals = plsc.load_gather(dv, [plane, wv, lane], mask=lmask)
                ins = (ixi[flt] != 0) & iyt
                res = jnp.where(ins, vals, jnp.float32(0.0))
                plsc.store_scatter(ov, [wv, lane], res, mask=lmask)

            pltpu.sync_copy(ov, out_hbm.at[hr, pl.ds(w0, wch), :])
            return carry

        lax.fori_loop(0, rh, row, 0)

    return sck(rt, data)


def _crop_split_body(rt_ref, data_ref, out_ref, *, rows, width, n):
    i = pl.program_id(0)
    x1 = rt_ref[0:1, :].reshape(1, 1, n)
    y1 = rt_ref[1:2, :].reshape(1, 1, n)
    x2 = rt_ref[2:3, :].reshape(1, 1, n)
    y2 = rt_ref[3:4, :].reshape(1, 1, n)
    wc = (x2 - x1) * 0.5
    hc = (y2 - y1) * 0.5

    ww = lax.broadcasted_iota(jnp.int32, (1, width, 1), 1).astype(jnp.float32)
    h0 = (i * rows).astype(jnp.float32)
    hh = lax.broadcasted_iota(jnp.int32, (rows, 1, 1), 0).astype(jnp.float32) + h0

    sx = ((ww - x1) / wc) >= 1.0
    sy = ((hh - y1) / hc) >= 1.0
    ins_x = (ww >= x1) & (ww <= x2)
    ins_y = (hh >= y1) & (hh <= y2)

    d0 = data_ref[0]
    d1 = data_ref[1]
    d2 = data_ref[2]
    d3 = data_ref[3]
    low = jnp.where(sx, d1, d0)
    high = jnp.where(sx, d3, d2)
    sel = jnp.where(sy, high, low)
    out_ref[...] = jnp.where(ins_x & ins_y, sel, jnp.float32(0.0))


def kernel(data, rois):
    return _sc_crop_split(data, rois)


def _tc_crop_split(data, rois):
    cc, h, w, n = data.shape
    rt = rois.T
    rows = _ROWS
    grid = (h // rows,)
    body = functools.partial(_crop_split_body, rows=rows, width=w, n=n)
    return pl.pallas_call(
        body,
        grid=grid,
        in_specs=[
            pl.BlockSpec((cc, n), lambda i: (0, 0)),
            pl.BlockSpec((cc, rows, w, n), lambda i: (0, i, 0, 0)),
        ],
        out_specs=pl.BlockSpec((rows, w, n), lambda i: (i, 0, 0)),
        out_shape=jax.ShapeDtypeStruct((h, w, n), data.dtype),
    )(rt, data)


# speedup vs baseline: 1.6444x; 1.6444x over previous
import functools

import jax
import jax.numpy as jnp
from jax import lax
from jax.experimental import pallas as pl
from jax.experimental.pallas import tpu as pltpu
from jax.experimental.pallas import tpu_sc as plsc

_C = 2
_ROWS = 10

_NC, _NS, _L = 2, 16, 16
_CCH = 8
_RBANDS = 4
_NG = 19
_NPAD = _NG * _L


def _sc_crop_split(data, rois):
    cc, h, w, n = data.shape
    wch = w // _CCH
    rh = h // _RBANDS
    rt = jnp.pad(rois.T, ((0, 0), (0, _NPAD - n)))
    mesh = plsc.VectorSubcoreMesh(
        core_axis_name="c", subcore_axis_name="s",
        num_cores=_NC, num_subcores=_NS)

    @functools.partial(
        pl.kernel,
        out_type=jax.ShapeDtypeStruct((h, w, n), data.dtype),
        mesh=mesh,
        compiler_params=pltpu.CompilerParams(
            use_tc_tiling_on_sc=False, needs_layout_passes=False),
        scratch_types=[
            pltpu.VMEM((_NPAD,), jnp.float32),
            pltpu.VMEM((_NPAD,), jnp.float32),
            pltpu.VMEM((_NPAD,), jnp.float32),
            pltpu.VMEM((_NPAD,), jnp.float32),
            pltpu.VMEM((_NPAD,), jnp.float32),
            pltpu.VMEM((_NPAD,), jnp.float32),
            pltpu.VMEM((cc, wch, n), jnp.float32),
            pltpu.VMEM((cc, wch, n), jnp.float32),
            pltpu.VMEM((wch, n), jnp.float32),
            pltpu.VMEM((wch, n), jnp.float32),
            pltpu.SemaphoreType.DMA,
            pltpu.SemaphoreType.DMA,
            pltpu.SemaphoreType.DMA,
            pltpu.SemaphoreType.DMA,
        ],
    )
    def sck(rt_hbm, data_hbm, out_hbm,
            x1v, y1v, x2v, y2v, xmv, ymv, dv0, dv1, ov0, ov1,
            sem_in0, sem_in1, sem_out0, sem_out1):
        cid = lax.axis_index("c")
        sid = lax.axis_index("s")
        wid = sid * _NC + cid
        w0 = (wid % _CCH) * wch
        h0 = (wid // _CCH) * rh

        pltpu.sync_copy(rt_hbm.at[0], x1v)
        pltpu.sync_copy(rt_hbm.at[1], y1v)
        pltpu.sync_copy(rt_hbm.at[2], x2v)
        pltpu.sync_copy(rt_hbm.at[3], y2v)

        def initg(g, carry):
            sl = pl.ds(g * _L, _L)
            x1g = x1v[sl]
            x2g = x2v[sl]
            xmv[sl] = x1g + (x2g - x1g) * 0.5
            y1g = y1v[sl]
            y2g = y2v[sl]
            ymv[sl] = y1g + (y2g - y1g) * 0.5
            return carry

        lax.fori_loop(0, _NG, initg, 0)

        i32_one = jnp.ones((_L,), jnp.int32)
        i32_zero = jnp.zeros((_L,), jnp.int32)
        w0f = w0.astype(jnp.float32)

        def in_copies(hr, dbuf, sem):
            return [pltpu.make_async_copy(
                data_hbm.at[p, hr, pl.ds(w0, wch), :], dbuf.at[p], sem)
                for p in range(4)]

        def out_copy(hr, obuf, sem):
            return pltpu.make_async_copy(
                obuf, out_hbm.at[hr, pl.ds(w0, wch), :], sem)

        def compute_row(hr, dv, ov):
            hfv = jnp.zeros((_L,), jnp.float32) + hr.astype(jnp.float32)

            def gbody(g, carry):
                sl = pl.ds(g * _L, _L)
                sy = hfv >= ymv[sl]
                iy = (hfv >= y1v[sl]) & (hfv <= y2v[sl])
                x1g = x1v[sl]
                x2g = x2v[sl]
                xmg = xmv[sl]
                for wi in range(wch):
                    wfv = jnp.zeros((_L,), jnp.float32) + (w0f + float(wi))
                    sx = wfv >= xmg
                    d0 = dv[0, wi, sl]
                    d1 = dv[1, wi, sl]
                    d2 = dv[2, wi, sl]
                    d3 = dv[3, wi, sl]
                    low = jnp.where(sx, d1, d0)
                    high = jnp.where(sx, d3, d2)
                    sel = jnp.where(sy, high, low)
                    ins = (wfv >= x1g) & (wfv <= x2g) & iy
                    ov[wi, sl] = jnp.where(ins, sel, jnp.float32(0.0))
                return carry

            lax.fori_loop(0, _NG - 1, gbody, 0)

            gt = _NG - 1
            slt = pl.ds(gt * _L, _L)
            lane = gt * _L + lax.iota(jnp.int32, _L)
            lmask = lane < n
            syt = hfv >= ymv[slt]
            sy2 = jnp.where(syt, i32_one + i32_one, i32_zero)
            iyt = (hfv >= y1v[slt]) & (hfv <= y2v[slt])
            x1t = x1v[slt]
            x2t = x2v[slt]
            xmt = xmv[slt]
            for wi in range(wch):
                wfv = jnp.zeros((_L,), jnp.float32) + (w0f + float(wi))
                plane = jnp.where(wfv >= xmt, i32_one, i32_zero) + sy2
                wv = jnp.full((_L,), wi, jnp.int32)
                vals = plsc.load_gather(dv, [plane, wv, lane], mask=lmask)
                ins = (wfv >= x1t) & (wfv <= x2t) & iyt
                res = jnp.where(ins, vals, jnp.float32(0.0))
                plsc.store_scatter(ov, [wv, lane], res, mask=lmask)

        for c in in_copies(h0, dv0, sem_in0):
            c.start()

        def rowpair(r2, carry):
            hr0 = h0 + 2 * r2
            hr1 = hr0 + 1
            for c in in_copies(hr1, dv1, sem_in1):
                c.start()
            for c in in_copies(hr0, dv0, sem_in0):
                c.wait()

            @pl.when(r2 >= 1)
            def _():
                out_copy(hr0 - 2, ov0, sem_out0).wait()

            compute_row(hr0, dv0, ov0)
            out_copy(hr0, ov0, sem_out0).start()

            @pl.when(r2 < rh // 2 - 1)
            def _():
                for c in in_copies(hr0 + 2, dv0, sem_in0):
                    c.start()

            for c in in_copies(hr1, dv1, sem_in1):
                c.wait()

            @pl.when(r2 >= 1)
            def _():
                out_copy(hr1 - 2, ov1, sem_out1).wait()

            compute_row(hr1, dv1, ov1)
            out_copy(hr1, ov1, sem_out1).start()
            return carry

        lax.fori_loop(0, rh // 2, rowpair, 0)
        out_copy(h0 + rh - 2, ov0, sem_out0).wait()
        out_copy(h0 + rh - 1, ov1, sem_out1).wait()

    return sck(rt, data)


def _crop_split_body(rt_ref, data_ref, out_ref, *, rows, width, n):
    i = pl.program_id(0)
    x1 = rt_ref[0:1, :].reshape(1, 1, n)
    y1 = rt_ref[1:2, :].reshape(1, 1, n)
    x2 = rt_ref[2:3, :].reshape(1, 1, n)
    y2 = rt_ref[3:4, :].reshape(1, 1, n)
    wc = (x2 - x1) * 0.5
    hc = (y2 - y1) * 0.5

    ww = lax.broadcasted_iota(jnp.int32, (1, width, 1), 1).astype(jnp.float32)
    h0 = (i * rows).astype(jnp.float32)
    hh = lax.broadcasted_iota(jnp.int32, (rows, 1, 1), 0).astype(jnp.float32) + h0

    sx = ((ww - x1) / wc) >= 1.0
    sy = ((hh - y1) / hc) >= 1.0
    ins_x = (ww >= x1) & (ww <= x2)
    ins_y = (hh >= y1) & (hh <= y2)

    d0 = data_ref[0]
    d1 = data_ref[1]
    d2 = data_ref[2]
    d3 = data_ref[3]
    low = jnp.where(sx, d1, d0)
    high = jnp.where(sx, d3, d2)
    sel = jnp.where(sy, high, low)
    out_ref[...] = jnp.where(ins_x & ins_y, sel, jnp.float32(0.0))


def kernel(data, rois):
    return _sc_crop_split(data, rois)


def _tc_crop_split(data, rois):
    cc, h, w, n = data.shape
    rt = rois.T
    rows = _ROWS
    grid = (h // rows,)
    body = functools.partial(_crop_split_body, rows=rows, width=w, n=n)
    return pl.pallas_call(
        body,
        grid=grid,
        in_specs=[
            pl.BlockSpec((cc, n), lambda i: (0, 0)),
            pl.BlockSpec((cc, rows, w, n), lambda i: (0, i, 0, 0)),
        ],
        out_specs=pl.BlockSpec((rows, w, n), lambda i: (i, 0, 0)),
        out_shape=jax.ShapeDtypeStruct((h, w, n), data.dtype),
    )(rt, data)


# speedup vs baseline: 12.8888x; 7.8379x over previous
import functools

import jax
import jax.numpy as jnp
from jax import lax
from jax.experimental import pallas as pl

_ROWS = 8


def _crop_split_body(rt_ref, data_ref, out_ref, *, rows, width, n):
    i = pl.program_id(0)
    x1 = rt_ref[0:1, :].reshape(1, 1, n)
    y1 = rt_ref[1:2, :].reshape(1, 1, n)
    x2 = rt_ref[2:3, :].reshape(1, 1, n)
    y2 = rt_ref[3:4, :].reshape(1, 1, n)
    wc = (x2 - x1) * 0.5
    hc = (y2 - y1) * 0.5

    ww = lax.broadcasted_iota(jnp.int32, (1, width, 1), 1).astype(jnp.float32)
    h0 = (i * rows).astype(jnp.float32)
    hh = lax.broadcasted_iota(jnp.int32, (rows, 1, 1), 0).astype(jnp.float32) + h0

    sx = ((ww - x1) / wc) >= 1.0
    sy = ((hh - y1) / hc) >= 1.0
    ins_x = (ww >= x1) & (ww <= x2)
    ins_y = (hh >= y1) & (hh <= y2)

    d0 = data_ref[0]
    d1 = data_ref[1]
    d2 = data_ref[2]
    d3 = data_ref[3]
    low = jnp.where(sx, d1, d0)
    high = jnp.where(sx, d3, d2)
    sel = jnp.where(sy, high, low)
    out_ref[...] = jnp.where(ins_x & ins_y, sel, jnp.float32(0.0))


def kernel(data, rois):
    cc, h, w, n = data.shape
    rt = rois.T
    rows = _ROWS
    grid = (h // rows,)
    body = functools.partial(_crop_split_body, rows=rows, width=w, n=n)
    return pl.pallas_call(
        body,
        grid=grid,
        in_specs=[
            pl.BlockSpec((cc, n), lambda i: (0, 0)),
            pl.BlockSpec((cc, rows, w, n), lambda i: (0, i, 0, 0)),
        ],
        out_specs=pl.BlockSpec((rows, w, n), lambda i: (i, 0, 0)),
        out_shape=jax.ShapeDtypeStruct((h, w, n), data.dtype),
    )(rt, data)
